# Initial kernel scaffold; baseline (speedup 1.0000x reference)
import jax
import jax.numpy as jnp
from jax.experimental import pallas as pl


def kernel(x, edge_index, W1, b1, W2, b2):
    raise NotImplementedError("write your pallas kernel here")


# speedup vs baseline: 38.8463x; 38.8463x over previous
import functools

import jax
import jax.numpy as jnp
from jax import lax
from jax.experimental import pallas as pl
from jax.experimental.pallas import tpu as pltpu
from jax.experimental.pallas import tpu_sc as plsc

NC = 2
NS = 16
LANES = 16
NW = NC * NS
K = 80
GRP = 5

N = 10000
H = 16
R_PAD = 10112
ROWS_TILE = R_PAD // NS


def _mesh():
    return plsc.VectorSubcoreMesh(core_axis_name="c", subcore_axis_name="s")


def _zero_fill(buf, rows):
    zvec = jnp.zeros((LANES,), jnp.float32)

    def body(i, carry):
        buf[i, :] = zvec
        return carry

    lax.fori_loop(0, rows, body, 0)


def _make_deg_kernel(n_chunks):
    @functools.partial(
        pl.kernel,
        mesh=_mesh(),
        out_type=jax.ShapeDtypeStruct((NC, R_PAD, H), jnp.float32),
        compiler_params=pltpu.CompilerParams(use_tc_tiling_on_sc=False),
        scratch_types=[
            pltpu.VMEM((n_chunks, K), jnp.int32),
            pltpu.VMEM((ROWS_TILE, H), jnp.float32),
            pltpu.VMEM((K, H), jnp.float32),
            pltpu.VMEM_SHARED((R_PAD, H), jnp.float32),
        ],
    )
    def deg_kernel(dst_hbm, out_hbm, dst_v, zbuf, ones_v, acc):
        c = lax.axis_index("c")
        s = lax.axis_index("s")
        wid = s * NC + c
        pltpu.sync_copy(dst_hbm.at[wid], dst_v)

        ovec = jnp.full((LANES,), 1.0, jnp.float32)

        def fill(i, carry):
            ones_v[i, :] = ovec
            return carry

        lax.fori_loop(0, K, fill, 0)

        _zero_fill(zbuf, ROWS_TILE)
        pltpu.sync_copy(zbuf, acc.at[pl.ds(s * ROWS_TILE, ROWS_TILE)])
        plsc.subcore_barrier()

        def body(j, carry):
            pltpu.sync_copy(ones_v, acc.at[dst_v.at[j]], add=True)
            return carry

        lax.fori_loop(0, n_chunks, body, 0)
        plsc.subcore_barrier()

        pltpu.sync_copy(acc.at[pl.ds(s * ROWS_TILE, ROWS_TILE)], zbuf)
        pltpu.sync_copy(zbuf, out_hbm.at[c, pl.ds(s * ROWS_TILE, ROWS_TILE)])

    return deg_kernel


def _make_agg_kernel(n_chunks):
    n_grp = n_chunks // GRP

    @functools.partial(
        pl.kernel,
        mesh=_mesh(),
        out_type=jax.ShapeDtypeStruct((NC, R_PAD, H), jnp.float32),
        compiler_params=pltpu.CompilerParams(use_tc_tiling_on_sc=False),
        scratch_types=[
            pltpu.VMEM((n_chunks, K), jnp.int32),
            pltpu.VMEM((n_chunks, K), jnp.int32),
            pltpu.VMEM((ROWS_TILE, H), jnp.float32),
            pltpu.VMEM((K, H), jnp.float32),
            pltpu.VMEM((K, H), jnp.float32),
            pltpu.VMEM((K, H), jnp.float32),
            pltpu.VMEM((K, H), jnp.float32),
            pltpu.VMEM((K, H), jnp.float32),
            pltpu.VMEM_SHARED((R_PAD, H), jnp.float32),
            pltpu.SemaphoreType.DMA,
        ],
    )
    def agg_kernel(t_hbm, src_hbm, dst_hbm, out_hbm,
                   src_v, dst_v, zbuf, b0, b1, b2, b3, b4, acc, sem):
        bufs = (b0, b1, b2, b3, b4)
        c = lax.axis_index("c")
        s = lax.axis_index("s")
        wid = s * NC + c
        pltpu.sync_copy(src_hbm.at[wid], src_v)
        pltpu.sync_copy(dst_hbm.at[wid], dst_v)

        _zero_fill(zbuf, ROWS_TILE)
        pltpu.sync_copy(zbuf, acc.at[pl.ds(s * ROWS_TILE, ROWS_TILE)])
        plsc.subcore_barrier()

        def body(g, carry):
            base = g * GRP
            handles = []
            for b in range(GRP):
                handles.append(
                    pltpu.async_copy(t_hbm.at[src_v.at[base + b]], bufs[b], sem))
            for b in range(GRP):
                handles[b].wait()
            for b in range(GRP):
                pltpu.sync_copy(bufs[b], acc.at[dst_v.at[base + b]], add=True)
            return carry

        lax.fori_loop(0, n_grp, body, 0)
        plsc.subcore_barrier()

        pltpu.sync_copy(acc.at[pl.ds(s * ROWS_TILE, ROWS_TILE)], zbuf)
        pltpu.sync_copy(zbuf, out_hbm.at[c, pl.ds(s * ROWS_TILE, ROWS_TILE)])

    return agg_kernel


_BLK = 1000


def _tc1_body(degp, x, w1, t1_out, dinv_out):
    deg = degp[0] + degp[1] + 1.0
    dinv = lax.rsqrt(deg)
    hw = jnp.dot(x[...], w1[...], preferred_element_type=jnp.float32)
    t1_out[...] = hw * dinv
    dinv_out[...] = dinv


def _tc2_body(s1, t1, dinv, b1, t2_out):
    agg = (s1[0] + s1[1] + t1[...]) * dinv[...] + b1[...]
    h = jnp.maximum(agg, 0.0)
    t2_out[...] = h * dinv[...]


def _tc3_body(s2, t2, dinv, w2, b2, out):
    agg = (s2[0] + s2[1] + t2[...]) * dinv[...]
    z = jnp.dot(agg, w2[...], preferred_element_type=jnp.float32) + b2[...]
    m = jnp.max(z, axis=1, keepdims=True)
    e = jnp.exp(z - m)
    lse = jnp.log(jnp.sum(e, axis=1, keepdims=True))
    out[...] = z - m - lse


def _stacked_spec():
    return pl.BlockSpec((NC, _BLK, H), lambda i: (0, i, 0))


def _rows_spec(w):
    return pl.BlockSpec((_BLK, w), lambda i: (i, 0))


def _full_spec(shape):
    return pl.BlockSpec(shape, lambda i: tuple(0 for _ in shape))


def _tc1(degp, x, w1):
    return pl.pallas_call(
        _tc1_body,
        grid=(N // _BLK,),
        in_specs=[_stacked_spec(), _rows_spec(128), _full_spec((128, H))],
        out_specs=[_rows_spec(H), _rows_spec(H)],
        out_shape=[jax.ShapeDtypeStruct((N, H), jnp.float32),
                   jax.ShapeDtypeStruct((N, H), jnp.float32)],
    )(degp, x, w1)


def _tc2(s1, t1, dinv, b1r):
    return pl.pallas_call(
        _tc2_body,
        grid=(N // _BLK,),
        in_specs=[_stacked_spec(), _rows_spec(H), _rows_spec(H),
                  _full_spec((1, H))],
        out_specs=_rows_spec(H),
        out_shape=jax.ShapeDtypeStruct((N, H), jnp.float32),
    )(s1, t1, dinv, b1r)


def _tc3(s2, t2, dinv, w2, b2r, n_cls):
    return pl.pallas_call(
        _tc3_body,
        grid=(N // _BLK,),
        in_specs=[_stacked_spec(), _rows_spec(H), _rows_spec(H),
                  _full_spec((H, n_cls)), _full_spec((1, n_cls))],
        out_specs=_rows_spec(n_cls),
        out_shape=jax.ShapeDtypeStruct((N, n_cls), jnp.float32),
    )(s2, t2, dinv, w2, b2r)


def kernel(x, edge_index, W1, b1, W2, b2):
    n_edges = edge_index.shape[1]
    per_worker = n_edges // NW
    n_chunks = per_worker // K

    src3 = edge_index[0].reshape(NW, n_chunks, K)
    dst3 = edge_index[1].reshape(NW, n_chunks, K)

    deg_k = _make_deg_kernel(n_chunks)
    agg_k = _make_agg_kernel(n_chunks)

    degp = deg_k(dst3)
    degp = degp[:, :N, :]

    t1, dinv = _tc1(degp, x, W1)

    s1 = agg_k(t1, src3, dst3)[:, :N, :]
    t2 = _tc2(s1, t1, dinv, b1.reshape(1, H))

    s2 = agg_k(t2, src3, dst3)[:, :N, :]
    n_cls = W2.shape[1]
    out = _tc3(s2, t2, dinv, W2, b2.reshape(1, n_cls), n_cls)
    return out
